# baseline (speedup 1.0000x reference)
import jax
import jax.numpy as jnp
from jax import lax
from jax.experimental import pallas as pl
from jax.experimental.pallas import tpu as pltpu, tpu_sc as plsc

_NC, _NS = 2, 16
_NW = _NC * _NS
_CH = 4
_NBUF = 6
_LOOKAHEAD = 3


def _sc_body(x, tab, out, xb, tb, *sems):
    sx, st, so = sems[0:_NBUF], sems[_NBUF:2 * _NBUF], sems[2 * _NBUF:3 * _NBUF]
    wid = lax.axis_index("s") * _NC + lax.axis_index("c")
    seq_len, batch, d_model = x.shape
    spw = seq_len // _NW
    n_chunks = spw // _CH
    s0 = wid * spw

    def in_start(c, b):
        off = s0 + c * _CH
        pltpu.make_async_copy(x.at[pl.ds(off, _CH)], xb.at[b], sx[b]).start()
        pltpu.make_async_copy(tab.at[pl.ds(off, _CH)], tb.at[b], st[b]).start()

    def in_wait(c, b):
        off = s0 + c * _CH
        pltpu.make_async_copy(x.at[pl.ds(off, _CH)], xb.at[b], sx[b]).wait()
        pltpu.make_async_copy(tab.at[pl.ds(off, _CH)], tb.at[b], st[b]).wait()

    def out_start(c, b):
        off = s0 + c * _CH
        pltpu.make_async_copy(xb.at[b], out.at[pl.ds(off, _CH)], so[b]).start()

    def out_wait(c, b):
        off = s0 + c * _CH
        pltpu.make_async_copy(xb.at[b], out.at[pl.ds(off, _CH)], so[b]).wait()

    def compute(b):
        @pl.loop(0, _CH)
        def _s(s):
            @pl.loop(0, d_model // 16, unroll=4)
            def _j(j):
                sl = pl.ds(j * 16, 16)
                tv = tb[b, s, sl]
                for bb in range(batch):
                    plsc.addupdate(xb.at[b, s, bb, sl], tv)

    for c in range(_LOOKAHEAD):
        in_start(c, c % _NBUF)

    for c in range(n_chunks):
        b = c % _NBUF
        in_wait(c, b)
        compute(b)
        out_start(c, b)
        f = c + _LOOKAHEAD
        if f < n_chunks:
            bf = f % _NBUF
            if f - _NBUF >= 0:
                out_wait(f - _NBUF, bf)
            in_start(f, bf)

    for c in range(max(0, n_chunks - _NBUF), n_chunks):
        out_wait(c, c % _NBUF)


def kernel(x, table):
    seq_len, batch, d_model = x.shape
    mesh = plsc.VectorSubcoreMesh(core_axis_name="c", subcore_axis_name="s")
    f = pl.kernel(
        _sc_body,
        out_type=jax.ShapeDtypeStruct(x.shape, x.dtype),
        mesh=mesh,
        scratch_types=[
            pltpu.VMEM((_NBUF, _CH, batch, d_model), jnp.float32),
            pltpu.VMEM((_NBUF, _CH, d_model), jnp.float32),
        ] + [pltpu.SemaphoreType.DMA] * (3 * _NBUF),
    )
    return f(x, table[:seq_len])


# speedup vs baseline: 1.0119x; 1.0119x over previous
import jax
import jax.numpy as jnp
from jax import lax
from jax.experimental import pallas as pl
from jax.experimental.pallas import tpu as pltpu, tpu_sc as plsc

_NC, _NS = 2, 16
_NW = _NC * _NS
_CH = 8
_NBUF = 3
_LOOKAHEAD = 2


def _sc_body(x, tab, out, xb, tb, *sems):
    sx, st, so = sems[0:_NBUF], sems[_NBUF:2 * _NBUF], sems[2 * _NBUF:3 * _NBUF]
    wid = lax.axis_index("s") * _NC + lax.axis_index("c")
    seq_len, batch, d_model = x.shape
    spw = seq_len // _NW
    n_chunks = spw // _CH
    s0 = wid * spw

    def in_start(c, b):
        off = s0 + c * _CH
        pltpu.make_async_copy(x.at[pl.ds(off, _CH)], xb.at[b], sx[b]).start()
        pltpu.make_async_copy(tab.at[pl.ds(off, _CH)], tb.at[b], st[b]).start()

    def in_wait(c, b):
        off = s0 + c * _CH
        pltpu.make_async_copy(x.at[pl.ds(off, _CH)], xb.at[b], sx[b]).wait()
        pltpu.make_async_copy(tab.at[pl.ds(off, _CH)], tb.at[b], st[b]).wait()

    def out_start(c, b):
        off = s0 + c * _CH
        pltpu.make_async_copy(xb.at[b], out.at[pl.ds(off, _CH)], so[b]).start()

    def out_wait(c, b):
        off = s0 + c * _CH
        pltpu.make_async_copy(xb.at[b], out.at[pl.ds(off, _CH)], so[b]).wait()

    def compute(b):
        @pl.loop(0, _CH)
        def _s(s):
            @pl.loop(0, d_model // 16, unroll=4)
            def _j(j):
                sl = pl.ds(j * 16, 16)
                tv = tb[b, s, sl]
                for bb in range(batch):
                    plsc.addupdate(xb.at[b, s, bb, sl], tv)

    for c in range(_LOOKAHEAD):
        in_start(c, c % _NBUF)

    for c in range(n_chunks):
        b = c % _NBUF
        in_wait(c, b)
        compute(b)
        out_start(c, b)
        f = c + _LOOKAHEAD
        if f < n_chunks:
            bf = f % _NBUF
            if f - _NBUF >= 0:
                out_wait(f - _NBUF, bf)
            in_start(f, bf)

    for c in range(max(0, n_chunks - _NBUF), n_chunks):
        out_wait(c, c % _NBUF)


def kernel(x, table):
    seq_len, batch, d_model = x.shape
    mesh = plsc.VectorSubcoreMesh(core_axis_name="c", subcore_axis_name="s")
    f = pl.kernel(
        _sc_body,
        out_type=jax.ShapeDtypeStruct(x.shape, x.dtype),
        mesh=mesh,
        scratch_types=[
            pltpu.VMEM((_NBUF, _CH, batch, d_model), jnp.float32),
            pltpu.VMEM((_NBUF, _CH, d_model), jnp.float32),
        ] + [pltpu.SemaphoreType.DMA] * (3 * _NBUF),
    )
    return f(x, table[:seq_len])
